# Initial kernel scaffold; baseline (speedup 1.0000x reference)
import functools

import jax
import jax.numpy as jnp
from jax import lax
from jax.experimental import pallas as pl
from jax.experimental.pallas import tpu as pltpu
from jax.experimental.pallas import tpu_sc as plsc


def _build(total, emb, chunk):
    info = plsc.get_sparse_core_info()
    nc, ns = info.num_cores, info.num_subcores
    nw = nc * ns
    per_w = total // nw
    assert total % nw == 0
    assert per_w % chunk == 0
    nchunk = per_w // chunk
    mesh = plsc.VectorSubcoreMesh(core_axis_name="c", subcore_axis_name="s")

    @functools.partial(
        pl.kernel,
        mesh=mesh,
        out_type=jax.ShapeDtypeStruct((total, emb), jnp.float32),
        scratch_types=[
            pltpu.VMEM((nchunk, chunk), jnp.int32),
            pltpu.VMEM((chunk, emb), jnp.float32),
            pltpu.SemaphoreType.DMA,
        ],
    )
    def k(table_hbm, idx_hbm, out_hbm, idx_v, rows_v, sem):
        c = lax.axis_index("c")
        s = lax.axis_index("s")
        wid = s * nc + c
        base = wid * per_w
        pltpu.sync_copy(idx_hbm.at[wid], idx_v)

        @pl.loop(0, nchunk)
        def _(g):
            pltpu.async_copy(table_hbm.at[idx_v.at[g]], rows_v, sem).wait()
            pltpu.sync_copy(rows_v, out_hbm.at[pl.ds(base + g * chunk, chunk)])

    return k, nw, nchunk


def kernel(pos, weight):
    b, s = pos.shape
    v, emb = weight.shape
    total = b * s
    chunk = 512
    k, nw, nchunk = _build(total, emb, chunk)
    idx = pos.reshape(-1).astype(jnp.int32).reshape(nw, nchunk, chunk)
    out = k(weight, idx)
    return out.reshape(b, s, emb)


# speedup vs baseline: 4.9050x; 4.9050x over previous
import functools

import jax
import jax.numpy as jnp
from jax import lax
from jax.experimental import pallas as pl
from jax.experimental.pallas import tpu as pltpu
from jax.experimental.pallas import tpu_sc as plsc


CHUNK = 128
K = 8


def _build(total, emb):
    info = plsc.get_sparse_core_info()
    nc, ns = info.num_cores, info.num_subcores
    nw = nc * ns
    per_w = total // nw
    assert total % nw == 0
    group = K * CHUNK
    assert per_w % group == 0
    ngroup = per_w // group
    nchunk = per_w // CHUNK
    mesh = plsc.VectorSubcoreMesh(core_axis_name="c", subcore_axis_name="s")

    @functools.partial(
        pl.kernel,
        mesh=mesh,
        out_type=jax.ShapeDtypeStruct((total, emb), jnp.float32),
        compiler_params=pltpu.CompilerParams(use_tc_tiling_on_sc=False),
        scratch_types=[
            pltpu.VMEM((nchunk, CHUNK), jnp.int32),
            pltpu.VMEM((group, emb), jnp.float32),
            pltpu.SemaphoreType.DMA,
        ],
    )
    def k(table_hbm, idx_hbm, out_hbm, idx_v, rows_v, sem):
        c = lax.axis_index("c")
        s = lax.axis_index("s")
        wid = s * nc + c
        base = wid * per_w
        pltpu.sync_copy(idx_hbm.at[wid], idx_v)

        @pl.loop(0, ngroup)
        def _(g):
            copies = []
            for j in range(K):
                cp = pltpu.async_copy(
                    table_hbm.at[idx_v.at[g * K + j]],
                    rows_v.at[pl.ds(j * CHUNK, CHUNK)],
                    sem,
                )
                copies.append(cp)
            for cp in copies:
                cp.wait()
            pltpu.sync_copy(rows_v, out_hbm.at[pl.ds(base + g * group, group)])

    return k, nw, nchunk


def kernel(pos, weight):
    b, s = pos.shape
    v, emb = weight.shape
    total = b * s
    k, nw, nchunk = _build(total, emb)
    idx = pos.reshape(-1).astype(jnp.int32).reshape(nw, nchunk, CHUNK)
    out = k(weight, idx)
    return out.reshape(b, s, emb)


# speedup vs baseline: 4.9439x; 1.0079x over previous
import functools

import jax
import jax.numpy as jnp
from jax import lax
from jax.experimental import pallas as pl
from jax.experimental.pallas import tpu as pltpu
from jax.experimental.pallas import tpu_sc as plsc

CHUNK = 128
K = 2
NS = 4
LAG = 2


def _build(total, emb):
    info = plsc.get_sparse_core_info()
    nc, ns = info.num_cores, info.num_subcores
    nw = nc * ns
    per_w = total // nw
    assert total % nw == 0
    group = K * CHUNK
    assert per_w % group == 0
    ngroup = per_w // group
    assert ngroup % NS == 0 and ngroup >= 3 * NS
    nchunk = per_w // CHUNK
    mesh = plsc.VectorSubcoreMesh(core_axis_name="c", subcore_axis_name="s")

    @functools.partial(
        pl.kernel,
        mesh=mesh,
        out_type=jax.ShapeDtypeStruct((total, emb), jnp.float32),
        compiler_params=pltpu.CompilerParams(use_tc_tiling_on_sc=False),
        scratch_types=[
            pltpu.VMEM((nchunk, CHUNK), jnp.int32),
            pltpu.VMEM((NS, group, emb), jnp.float32),
            pltpu.SemaphoreType.DMA,
            pltpu.SemaphoreType.DMA,
        ],
    )
    def k(table_hbm, idx_hbm, out_hbm, idx_v, rows_v, sem_g, sem_o):
        c = lax.axis_index("c")
        s = lax.axis_index("s")
        wid = s * nc + c
        base = wid * per_w
        pltpu.sync_copy(idx_hbm.at[wid], idx_v)

        def gather(q, slot):
            for j in range(K):
                yield pltpu.make_async_copy(
                    table_hbm.at[idx_v.at[q * K + j]],
                    rows_v.at[slot, pl.ds(j * CHUNK, CHUNK)],
                    sem_g,
                )

        def fire_gather(q, slot):
            for cp in gather(q, slot):
                cp.start()

        def wait_gather(q, slot):
            for cp in gather(q, slot):
                cp.wait()

        def out_cp(q, slot):
            return pltpu.make_async_copy(
                rows_v.at[slot],
                out_hbm.at[pl.ds(base + q * group, group)],
                sem_o,
            )

        def step(q, sl, refill, drain_out):
            wait_gather(q, sl)
            out_cp(q, sl).start()
            if refill:
                sl2 = (sl + NS - LAG) % NS
                if drain_out:
                    out_cp(q - LAG, sl2).wait()
                fire_gather(q + NS - LAG, sl2)

        for sl in range(NS):
            fire_gather(sl, sl)

        for sl in range(NS):
            step(sl, sl, refill=(sl + NS - LAG >= NS), drain_out=(sl >= LAG))

        @pl.loop(NS, ngroup - NS, step=NS)
        def _(g):
            for sl in range(NS):
                step(g + sl, sl, refill=True, drain_out=True)

        for sl in range(NS):
            q = ngroup - NS + sl
            step(q, sl, refill=(sl < LAG), drain_out=True)

        for sl in range(NS):
            out_cp(ngroup - NS + sl, sl).wait()

    return k, nw, nchunk


def kernel(pos, weight):
    b, s = pos.shape
    v, emb = weight.shape
    total = b * s
    k, nw, nchunk = _build(total, emb)
    idx = pos.reshape(-1).astype(jnp.int32).reshape(nw, nchunk, CHUNK)
    out = k(weight, idx)
    return out.reshape(b, s, emb)


# speedup vs baseline: 5.3641x; 1.0850x over previous
import functools

import jax
import jax.numpy as jnp
from jax import lax
from jax.experimental import pallas as pl
from jax.experimental.pallas import tpu as pltpu
from jax.experimental.pallas import tpu_sc as plsc

LANES = 16


def _build(bsz, seq, emb, padded):
    info = plsc.get_sparse_core_info()
    nc, ns = info.num_cores, info.num_subcores
    nw = nc * ns
    rows_w = bsz // nw
    assert bsz % nw == 0 and rows_w % 2 == 0 and rows_w >= 4
    seg = (seq // 128) * 128
    tail = seq - seg
    assert seg and tail and tail % 8 == 0
    sizes = (seg, tail)
    offs = (0, seg)
    nunits = 2 * rows_w
    mesh = plsc.VectorSubcoreMesh(core_axis_name="c", subcore_axis_name="s")

    @functools.partial(
        pl.kernel,
        mesh=mesh,
        out_type=jax.ShapeDtypeStruct((bsz, seq, emb), jnp.float32),
        scratch_types=[
            pltpu.VMEM((rows_w, seq), jnp.int32),
            pltpu.VMEM((seg, padded), jnp.float32),
            pltpu.VMEM((tail, padded), jnp.float32),
            pltpu.VMEM((seg, emb), jnp.float32),
            pltpu.VMEM((tail, emb), jnp.float32),
            pltpu.SemaphoreType.DMA,
            pltpu.SemaphoreType.DMA,
        ],
    )
    def k(table_hbm, pos_hbm, out_hbm, idx_v, ba0, ba1, bb0, bb1, sem_g, sem_o):
        bufa = (ba0, ba1)
        bufb = (bb0, bb1)
        c = lax.axis_index("c")
        s = lax.axis_index("s")
        wid = s * nc + c
        row0 = wid * rows_w
        pltpu.sync_copy(pos_hbm.at[pl.ds(row0, rows_w)], idx_v)

        def gather(u, p):
            return pltpu.make_async_copy(
                table_hbm.at[idx_v.at[u // 2, pl.ds(offs[p], sizes[p])]],
                bufa[p],
                sem_g,
            )

        def compact(p):
            a, b = bufa[p], bufb[p]

            @pl.loop(0, sizes[p])
            def _(r):
                for cc in range(emb // LANES):
                    b[r, pl.ds(cc * LANES, LANES)] = a[r, pl.ds(cc * LANES, LANES)]

        def out_cp(u, p):
            return pltpu.make_async_copy(
                bufb[p],
                out_hbm.at[row0 + u // 2, pl.ds(offs[p], sizes[p]), :],
                sem_o,
            )

        def step(u, p, drain_out, refill):
            gather(u, p).wait()
            compact(p)
            out_cp(u, p).start()
            if drain_out:
                out_cp(u - 1, 1 - p).wait()
            if refill:
                gather(u + 2, p).start()

        gather(0, 0).start()
        gather(1, 1).start()
        step(0, 0, drain_out=False, refill=True)
        step(1, 1, drain_out=True, refill=True)

        @pl.loop(2, nunits - 2, step=2)
        def _(u0):
            for p in range(2):
                step(u0 + p, p, drain_out=True, refill=True)

        for u in range(nunits - 2, nunits):
            step(u, u % 2, drain_out=True, refill=False)
        out_cp(nunits - 1, 1).wait()

    return k


def kernel(pos, weight):
    b, s = pos.shape
    v, emb = weight.shape
    padded = 128
    k = _build(b, s, emb, padded)
    table = jnp.pad(weight, ((0, 0), (0, padded - emb)))
    return k(table, pos.astype(jnp.int32))
